# Initial kernel scaffold; baseline (speedup 1.0000x reference)
import jax
import jax.numpy as jnp
from jax.experimental import pallas as pl


def kernel(text_vec, cand_vecs, W_ctx, W_cand):
    raise NotImplementedError("write your pallas kernel here")


# speedup vs baseline: 6.2196x; 6.2196x over previous
import functools

import jax
import jax.numpy as jnp
from jax.experimental import pallas as pl
from jax.experimental.pallas import tpu as pltpu

V, D, B, L = 1000, 64, 4096, 200
VP = 1024
LP = 256
BB = 256
CH = 8
BM = 512
BN = 512


def _encode_kernel(idx_ref, w_ref, out_ref, counts_ref):
    counts_ref[...] = jnp.zeros((BB, VP), jnp.float32)
    iota = jax.lax.broadcasted_iota(jnp.int32, (1, 1, VP), 2)

    def body(c, carry):
        chunk = idx_ref[pl.ds(c * CH, CH), :]
        onehot = (chunk[:, :, None] == iota).astype(jnp.float32)
        counts_ref[...] += jnp.sum(onehot, axis=0)
        return carry

    jax.lax.fori_loop(0, LP // CH, body, 0)
    out_ref[...] = jnp.dot(
        counts_ref[...], w_ref[0], preferred_element_type=jnp.float32
    ) * (1.0 / L)


def _score_kernel(a_ref, b_ref, out_ref):
    out_ref[...] = jax.lax.dot_general(
        a_ref[...], b_ref[...], (((1,), (1,)), ((), ())),
        preferred_element_type=jnp.float32)


@jax.jit
def kernel(text_vec, cand_vecs, W_ctx, W_cand):
    idx = jnp.concatenate([text_vec, cand_vecs], axis=0).astype(jnp.int32)
    idx = jnp.pad(idx, ((0, 0), (0, LP - L)), constant_values=VP - 1)
    idx_t = idx.T
    w = jnp.stack([
        jnp.pad(W_ctx, ((0, VP - V), (0, 0))),
        jnp.pad(W_cand, ((0, VP - V), (0, 0))),
    ])

    nblk = B // BB
    encs = pl.pallas_call(
        _encode_kernel,
        grid=(2 * nblk,),
        in_specs=[
            pl.BlockSpec((LP, BB), lambda i: (0, i)),
            pl.BlockSpec((1, VP, D), lambda i: (i // nblk, 0, 0)),
        ],
        out_specs=pl.BlockSpec((BB, D), lambda i: (i, 0)),
        out_shape=jax.ShapeDtypeStruct((2 * B, D), jnp.float32),
        scratch_shapes=[pltpu.VMEM((BB, VP), jnp.float32)],
    )(idx_t, w)

    ctx_enc, cand_enc = encs[:B], encs[B:]
    out = pl.pallas_call(
        _score_kernel,
        grid=(B // BM, B // BN),
        in_specs=[
            pl.BlockSpec((BM, D), lambda i, j: (i, 0)),
            pl.BlockSpec((BN, D), lambda i, j: (j, 0)),
        ],
        out_specs=pl.BlockSpec((BM, BN), lambda i, j: (i, j)),
        out_shape=jax.ShapeDtypeStruct((B, B), jnp.float32),
    )(ctx_enc, cand_enc)
    return out


# speedup vs baseline: 25.2175x; 4.0545x over previous
import functools

import jax
import jax.numpy as jnp
from jax import lax
from jax.experimental import pallas as pl
from jax.experimental.pallas import tpu as pltpu
from jax.experimental.pallas import tpu_sc as plsc

V, D, B, L = 1000, 64, 4096, 200
VP = 1024
NC, NS = 2, 16
NW = NC * NS
RPW = 2 * B // NW
RC = 64
NCHUNK = RPW // RC
BB = 256
BM = 512
BN = 512


def _hist_kernel(idx_hbm, counts_hbm, idx_v, hist_v):
    wid = lax.axis_index("s") * NC + lax.axis_index("c")
    lane = lax.iota(jnp.int32, 16)
    ones = jnp.ones((16,), jnp.float32)
    zeros = jnp.zeros((16,), jnp.float32)

    def zero_body(i, c):
        for u in range(4):
            hist_v[pl.ds((i * 4 + u) * 16, 16)] = zeros
        return c

    lax.fori_loop(0, RC * VP // 64, zero_body, 0)

    def scatter(value, add):
        def lbody(l, c):
            lsplat = jnp.full((16,), 0, jnp.int32) + l
            for g in range(RC // 16):
                rows = lane + (g * 16)
                iv = plsc.load_gather(idx_v, [rows * L + lsplat])
                flat = rows * VP + iv
                if add:
                    plsc.addupdate_scatter(hist_v, [flat], value)
                else:
                    plsc.store_scatter(hist_v, [flat], value)
            return c

        lax.fori_loop(0, L, lbody, 0)

    for chunk in range(NCHUNK):
        row_base = wid * RPW + chunk * RC
        pltpu.sync_copy(idx_hbm.at[pl.ds(row_base * L, RC * L)], idx_v)
        scatter(ones, add=True)
        pltpu.sync_copy(hist_v, counts_hbm.at[pl.ds(row_base * VP, RC * VP)])
        scatter(zeros, add=False)


def _encode_kernel(counts_ref, w_ref, out_ref):
    out_ref[...] = jnp.dot(
        counts_ref[...], w_ref[0], preferred_element_type=jnp.float32
    ) * (1.0 / L)


def _score_kernel(a_ref, b_ref, out_ref):
    out_ref[...] = lax.dot_general(
        a_ref[...], b_ref[...], (((1,), (1,)), ((), ())),
        preferred_element_type=jnp.float32)


@jax.jit
def kernel(text_vec, cand_vecs, W_ctx, W_cand):
    idx = jnp.concatenate([text_vec, cand_vecs], axis=0).astype(jnp.int32)
    w = jnp.stack([
        jnp.pad(W_ctx, ((0, VP - V), (0, 0))),
        jnp.pad(W_cand, ((0, VP - V), (0, 0))),
    ])

    hist_fn = pl.kernel(
        _hist_kernel,
        out_type=jax.ShapeDtypeStruct((2 * B * VP,), jnp.float32),
        mesh=plsc.VectorSubcoreMesh(
            core_axis_name="c", subcore_axis_name="s",
            num_cores=NC, num_subcores=NS),
        compiler_params=pltpu.CompilerParams(needs_layout_passes=False),
        scratch_types=[
            pltpu.VMEM((RC * L,), jnp.int32),
            pltpu.VMEM((RC * VP,), jnp.float32),
        ],
    )
    counts = hist_fn(idx.reshape(-1)).reshape(2 * B, VP)

    nblk = B // BB
    encs = pl.pallas_call(
        _encode_kernel,
        grid=(2 * nblk,),
        in_specs=[
            pl.BlockSpec((BB, VP), lambda i: (i, 0)),
            pl.BlockSpec((1, VP, D), lambda i: (i // nblk, 0, 0)),
        ],
        out_specs=pl.BlockSpec((BB, D), lambda i: (i, 0)),
        out_shape=jax.ShapeDtypeStruct((2 * B, D), jnp.float32),
    )(counts, w)

    ctx_enc, cand_enc = encs[:B], encs[B:]
    out = pl.pallas_call(
        _score_kernel,
        grid=(B // BM, B // BN),
        in_specs=[
            pl.BlockSpec((BM, D), lambda i, j: (i, 0)),
            pl.BlockSpec((BN, D), lambda i, j: (j, 0)),
        ],
        out_specs=pl.BlockSpec((BM, BN), lambda i, j: (i, j)),
        out_shape=jax.ShapeDtypeStruct((B, B), jnp.float32),
    )(ctx_enc, cand_enc)
    return out


# speedup vs baseline: 28.3414x; 1.1239x over previous
import functools

import jax
import jax.numpy as jnp
from jax import lax
from jax.experimental import pallas as pl
from jax.experimental.pallas import tpu as pltpu
from jax.experimental.pallas import tpu_sc as plsc

V, D, B, L = 1000, 64, 4096, 200
VP = 1024
NC, NS = 2, 16
NW = NC * NS
HALF = NW // 2
RPW = B // HALF
RC = 64
NCHUNK = RPW // RC
BB = 256
BM = 1024
BN = 1024


def _hist_kernel(text_hbm, cand_hbm, counts_hbm, idx_v, hist_v):
    wid = lax.axis_index("s") * NC + lax.axis_index("c")
    lane = lax.iota(jnp.int32, 16)
    ones = jnp.ones((16,), jnp.float32)
    zeros = jnp.zeros((16,), jnp.float32)

    def scatter(value, add):
        def lbody(l, c):
            lsplat = jnp.full((16,), 0, jnp.int32) + l
            for g in range(RC // 16):
                rows = lane + (g * 16)
                iv = plsc.load_gather(idx_v, [rows, lsplat])
                if add:
                    plsc.addupdate_scatter(hist_v, [rows, iv], value)
                else:
                    plsc.store_scatter(hist_v, [rows, iv], value)
            return c

        lax.fori_loop(0, L, lbody, 0)

    def zero_body(r, c):
        for u in range(VP // 16):
            hist_v[r, pl.ds(u * 16, 16)] = zeros
        return c

    def run(idx_hbm, side_base):
        lax.fori_loop(0, RC, zero_body, 0)
        for chunk in range(NCHUNK):
            row_base = (wid % HALF) * RPW + chunk * RC
            pltpu.sync_copy(idx_hbm.at[pl.ds(row_base, RC)], idx_v)
            scatter(ones, add=True)
            pltpu.sync_copy(
                hist_v, counts_hbm.at[pl.ds(side_base + row_base, RC)])
            scatter(zeros, add=False)

    @pl.when(wid < HALF)
    def _():
        run(text_hbm, 0)

    @pl.when(wid >= HALF)
    def _():
        run(cand_hbm, B)


def _encode_kernel(counts_ref, w_ref, out_ref):
    out_ref[...] = jnp.dot(
        counts_ref[...], w_ref[0], preferred_element_type=jnp.float32
    ) * (1.0 / L)


def _score_kernel(a_ref, b_ref, out_ref):
    out_ref[...] = lax.dot_general(
        a_ref[...], b_ref[...], (((1,), (1,)), ((), ())),
        preferred_element_type=jnp.float32)


@jax.jit
def kernel(text_vec, cand_vecs, W_ctx, W_cand):
    text_vec = text_vec.astype(jnp.int32)
    cand_vecs = cand_vecs.astype(jnp.int32)
    w = jnp.stack([
        jnp.pad(W_ctx, ((0, VP - V), (0, 0))),
        jnp.pad(W_cand, ((0, VP - V), (0, 0))),
    ])

    hist_fn = pl.kernel(
        _hist_kernel,
        out_type=jax.ShapeDtypeStruct((2 * B, VP), jnp.float32),
        mesh=plsc.VectorSubcoreMesh(
            core_axis_name="c", subcore_axis_name="s",
            num_cores=NC, num_subcores=NS),
        compiler_params=pltpu.CompilerParams(needs_layout_passes=False),
        scratch_types=[
            pltpu.VMEM((RC, L), jnp.int32),
            pltpu.VMEM((RC, VP), jnp.float32),
        ],
    )
    counts = hist_fn(text_vec, cand_vecs)

    nblk = B // BB
    encs = pl.pallas_call(
        _encode_kernel,
        grid=(2 * nblk,),
        in_specs=[
            pl.BlockSpec((BB, VP), lambda i: (i, 0)),
            pl.BlockSpec((1, VP, D), lambda i: (i // nblk, 0, 0)),
        ],
        out_specs=pl.BlockSpec((BB, D), lambda i: (i, 0)),
        out_shape=jax.ShapeDtypeStruct((2 * B, D), jnp.float32),
    )(counts, w)

    ctx_enc, cand_enc = encs[:B], encs[B:]
    out = pl.pallas_call(
        _score_kernel,
        grid=(B // BM, B // BN),
        in_specs=[
            pl.BlockSpec((BM, D), lambda i, j: (i, 0)),
            pl.BlockSpec((BN, D), lambda i, j: (j, 0)),
        ],
        out_specs=pl.BlockSpec((BM, BN), lambda i, j: (i, j)),
        out_shape=jax.ShapeDtypeStruct((B, B), jnp.float32),
    )(ctx_enc, cand_enc)
    return out
